# baseline (speedup 1.0000x reference)
import functools

import jax
import jax.numpy as jnp
from jax import lax
from jax.experimental import pallas as pl
from jax.experimental.pallas import tpu as pltpu
from jax.experimental.pallas import tpu_sc as plsc

EPS = 1e-7
B, N, K = 64, 2048, 16
F = 8
L = 16
NC, NS = 2, 16
NW = NC * NS
HB = B // 2
NBLK = N // L
KG = 8
NSWEEP = K // KG

_DNUMS = lax.GatherDimensionNumbers(
    offset_dims=(), collapsed_slice_dims=(0,), start_index_map=(0,))


def _bcast(vec, lane):
  idx = jnp.full((L, 1), lane, dtype=jnp.int32)
  return lax.gather(vec, idx, _DNUMS, slice_sizes=(1,),
                    mode=lax.GatherScatterMode.PROMISE_IN_BOUNDS)


def _make_sc_kernel(hb0):
  mesh = plsc.VectorSubcoreMesh(
      core_axis_name="c", subcore_axis_name="s", num_cores=NC,
      num_subcores=NS)

  @functools.partial(
      pl.kernel,
      mesh=mesh,
      compiler_params=pltpu.CompilerParams(needs_layout_passes=False),
      out_type=jax.ShapeDtypeStruct((HB * K, N), jnp.float32),
      scratch_types=[
          pltpu.VMEM((K, N), jnp.float32),
          pltpu.VMEM((F, N), jnp.float32),
          pltpu.VMEM((K, N), jnp.float32),
          pltpu.VMEM((2 * K * L,), jnp.float32),
          pltpu.SemaphoreType.DMA,
          pltpu.SemaphoreType.DMA,
          pltpu.SemaphoreType.DMA,
      ],
  )
  def sc_kernel(fracs_hbm, feats_hbm, out_hbm, fracs_v, feat_v, out_v,
                red_v, sem_f, sem_e, sem_o):
    wid = lax.axis_index("s") * NC + lax.axis_index("c")
    iota = lax.iota(jnp.int32, L)
    zero = jnp.zeros((L,), jnp.float32)
    b = hb0 + wid

    fcopy = pltpu.async_copy(fracs_hbm.at[pl.ds(b * K, K)], fracs_v, sem_f)
    ecopy = pltpu.async_copy(feats_hbm.at[pl.ds(b * F, F)], feat_v, sem_e)
    fcopy.wait()
    ecopy.wait()

    sums0 = []
    sums1 = []
    for s in range(NSWEEP):
      def swp(i, carry, s=s):
        a0 = list(carry[0])
        a1 = list(carry[1])
        n0 = i * L
        e16 = feat_v[0, pl.ds(n0, L)]
        eta16 = feat_v[1, pl.ds(n0, L)]
        ee16 = e16 * eta16
        for g in range(KG):
          x = fracs_v[s * KG + g, pl.ds(n0, L)]
          a0[g] = a0[g] + e16 * x
          a1[g] = a1[g] + ee16 * x
        return tuple(a0), tuple(a1)

      a0, a1 = lax.fori_loop(
          0, NBLK, swp, (tuple([zero] * KG), tuple([zero] * KG)))
      sums0.extend(a0)
      sums1.extend(a1)

    for k in range(K):
      red_v[pl.ds(k * L, L)] = sums0[k]
      red_v[pl.ds((K + k) * L, L)] = sums1[k]
    s0 = zero
    s1 = zero
    rowbase = iota * L
    for l in range(L):
      s0 = s0 + plsc.load_gather(red_v, [rowbase + l])
      s1 = s1 + plsc.load_gather(red_v, [rowbase + (K * L) + l])

    w = s1 / (s0 + EPS)
    w = jnp.where(jnp.abs(w) > 0.1, w, jnp.float32(500.0))

    rank = jnp.zeros((L,), jnp.int32)
    for j in range(K):
      bj = _bcast(w, j)
      before = (bj < w) | ((bj == w) & (iota > j))
      rank = rank + jnp.where(before, 1, 0)
    branks = [_bcast(rank, k) for k in range(K)]

    def blk2(i, c, branks=branks):
      n0 = i * L
      cols = n0 + iota
      rows = [fracs_v[k, pl.ds(n0, L)] for k in range(K)]
      for k in range(K):
        plsc.store_scatter(out_v, [branks[k], cols], rows[k])
      return c

    lax.fori_loop(0, NBLK, blk2, 0)
    pltpu.async_copy(out_v, out_hbm.at[pl.ds(wid * K, K)], sem_o).wait()

  return sc_kernel


_sc_kernel_a = _make_sc_kernel(0)
_sc_kernel_b = _make_sc_kernel(HB)


@jax.jit
def kernel(predicted_fracs, features):
  fracs_t = predicted_fracs.transpose(0, 2, 1).reshape(B * K, N)
  feats_t = features.transpose(0, 2, 1).reshape(B * F, N)
  out_a = _sc_kernel_a(fracs_t, feats_t)
  out_b = _sc_kernel_b(fracs_t, feats_t)
  out_t = jnp.concatenate([out_a, out_b], axis=0)
  return out_t.reshape(B, K, N).transpose(0, 2, 1)


# speedup vs baseline: 1.4324x; 1.4324x over previous
import functools

import jax
import jax.numpy as jnp
from jax import lax
from jax.experimental import pallas as pl
from jax.experimental.pallas import tpu as pltpu
from jax.experimental.pallas import tpu_sc as plsc

EPS = 1e-7
B, N, K = 64, 2048, 16
F = 8
L = 16
NC, NS = 2, 16
NW = NC * NS
BPW = B // NW
NBLK = N // L
KG = 8
NSWEEP = K // KG

_DNUMS = lax.GatherDimensionNumbers(
    offset_dims=(), collapsed_slice_dims=(0,), start_index_map=(0,))


def _bcast(vec, lane):
  idx = jnp.full((L, 1), lane, dtype=jnp.int32)
  return lax.gather(vec, idx, _DNUMS, slice_sizes=(1,),
                    mode=lax.GatherScatterMode.PROMISE_IN_BOUNDS)


def _make_sc_kernel():
  mesh = plsc.VectorSubcoreMesh(
      core_axis_name="c", subcore_axis_name="s", num_cores=NC,
      num_subcores=NS)

  @functools.partial(
      pl.kernel,
      mesh=mesh,
      compiler_params=pltpu.CompilerParams(needs_layout_passes=False),
      out_type=jax.ShapeDtypeStruct((B * K, N), jnp.float32),
      scratch_types=[
          pltpu.VMEM((K, N), jnp.float32),
          pltpu.VMEM((K, N), jnp.float32),
          pltpu.VMEM((2, N), jnp.float32),
          pltpu.VMEM((K, N), jnp.float32),
          pltpu.VMEM((2 * K * L,), jnp.float32),
          pltpu.SemaphoreType.DMA,
          pltpu.SemaphoreType.DMA,
          pltpu.SemaphoreType.DMA,
          pltpu.SemaphoreType.DMA,
          pltpu.SemaphoreType.DMA,
          pltpu.SemaphoreType.DMA,
      ],
  )
  def sc_kernel(fracs_hbm, feats_hbm, out_hbm, fracs_v0, fracs_v1,
                feat_v, out_v, red_v, sem_a0, sem_b0, sem_a1, sem_b1,
                sem_e, sem_o):
    wid = lax.axis_index("s") * NC + lax.axis_index("c")
    iota = lax.iota(jnp.int32, L)
    zero = jnp.zeros((L,), jnp.float32)

    fracs_bufs = (fracs_v0, fracs_v1)
    b0 = wid * BPW

    feat_copy = pltpu.async_copy(feats_hbm.at[pl.ds(b0 * F, 2)], feat_v,
                                 sem_e)
    chunk_sems = ((sem_a0, sem_b0), (sem_a1, sem_b1))
    chunk_copies = [
        [pltpu.async_copy(
            fracs_hbm.at[pl.ds((b0 + bi) * K + s * KG, KG)],
            fracs_bufs[bi].at[pl.ds(s * KG, KG)], chunk_sems[bi][s])
         for s in range(NSWEEP)]
        for bi in range(BPW)
    ]

    out_copy = None
    for bi in range(BPW):
      fracs_v = fracs_bufs[bi]
      feat_copy.wait()

      sums0 = []
      sums1 = []
      for s in range(NSWEEP):
        chunk_copies[bi][s].wait()

        def swp(i, carry, s=s):
          a0 = list(carry[0])
          a1 = list(carry[1])
          n0 = i * L
          e16 = feat_v[0, pl.ds(n0, L)]
          eta16 = feat_v[1, pl.ds(n0, L)]
          ee16 = e16 * eta16
          for g in range(KG):
            x = fracs_v[s * KG + g, pl.ds(n0, L)]
            a0[g] = a0[g] + e16 * x
            a1[g] = a1[g] + ee16 * x
          return tuple(a0), tuple(a1)

        a0, a1 = lax.fori_loop(
            0, NBLK, swp, (tuple([zero] * KG), tuple([zero] * KG)))
        sums0.extend(a0)
        sums1.extend(a1)

      for k in range(K):
        red_v[pl.ds(k * L, L)] = sums0[k]
        red_v[pl.ds((K + k) * L, L)] = sums1[k]
      s0 = zero
      s1 = zero
      rowbase = iota * L
      for l in range(L):
        s0 = s0 + plsc.load_gather(red_v, [rowbase + l])
        s1 = s1 + plsc.load_gather(red_v, [rowbase + (K * L) + l])

      if bi + 1 < BPW:
        feat_copy = pltpu.async_copy(
            feats_hbm.at[pl.ds((b0 + bi + 1) * F, 2)], feat_v, sem_e)

      w = s1 / (s0 + EPS)
      w = jnp.where(jnp.abs(w) > 0.1, w, jnp.float32(500.0))

      rank = jnp.zeros((L,), jnp.int32)
      for j in range(K):
        bj = _bcast(w, j)
        before = (bj < w) | ((bj == w) & (iota > j))
        rank = rank + jnp.where(before, 1, 0)
      branks = [_bcast(rank, k) for k in range(K)]

      if out_copy is not None:
        out_copy.wait()

      def blk2(i, c, branks=branks):
        n0 = i * L
        cols = n0 + iota
        rows = [fracs_v[k, pl.ds(n0, L)] for k in range(K)]
        for k in range(K):
          plsc.store_scatter(out_v, [branks[k], cols], rows[k])
        return c

      lax.fori_loop(0, NBLK, blk2, 0)
      out_copy = pltpu.async_copy(out_v, out_hbm.at[pl.ds((b0 + bi) * K, K)],
                                  sem_o)

    out_copy.wait()

  return sc_kernel


_sc_kernel = _make_sc_kernel()


@jax.jit
def kernel(predicted_fracs, features):
  fracs_t = predicted_fracs.transpose(0, 2, 1).reshape(B * K, N)
  feats_t = features.transpose(0, 2, 1).reshape(B * F, N)
  out_t = _sc_kernel(fracs_t, feats_t)
  return out_t.reshape(B, K, N).transpose(0, 2, 1)
